# baseline (speedup 1.0000x reference)
import jax
import jax.numpy as jnp
from jax.experimental import pallas as pl
from jax.experimental.pallas import tpu as pltpu

M, K, N = 2048, 4096, 4096
BN = 256


def _mm_kernel(a_ref, b_ref, o_ref):
    o_ref[...] = jnp.dot(a_ref[...], b_ref[...], preferred_element_type=jnp.float32)


def kernel(dense_a, dense_data):
    return pl.pallas_call(
        _mm_kernel,
        grid=(N // BN,),
        in_specs=[
            pl.BlockSpec((M, K), lambda n: (0, 0)),
            pl.BlockSpec((K, BN), lambda n: (0, n)),
        ],
        out_specs=pl.BlockSpec((M, BN), lambda n: (0, n)),
        out_shape=jax.ShapeDtypeStruct((M, N), jnp.float32),
        compiler_params=pltpu.CompilerParams(
            dimension_semantics=("arbitrary",),
        ),
    )(dense_a, dense_data)


# speedup vs baseline: 1.0103x; 1.0103x over previous
import jax
import jax.numpy as jnp
from jax.experimental import pallas as pl
from jax.experimental.pallas import tpu as pltpu

M, K, N = 2048, 4096, 4096
BK, BN = 512, 2048


def _mm_kernel(a_ref, b_ref, o_ref):
    k = pl.program_id(1)

    @pl.when(k == 0)
    def _init():
        o_ref[...] = jnp.zeros_like(o_ref)

    o_ref[...] += jnp.dot(a_ref[...], b_ref[...], preferred_element_type=jnp.float32)


def kernel(dense_a, dense_data):
    grid = (N // BN, K // BK)
    return pl.pallas_call(
        _mm_kernel,
        grid=grid,
        in_specs=[
            pl.BlockSpec((M, BK), lambda n, k: (0, k)),
            pl.BlockSpec((BK, BN), lambda n, k: (k, n)),
        ],
        out_specs=pl.BlockSpec((M, BN), lambda n, k: (0, n)),
        out_shape=jax.ShapeDtypeStruct((M, N), jnp.float32),
        compiler_params=pltpu.CompilerParams(
            dimension_semantics=("parallel", "arbitrary"),
        ),
    )(dense_a, dense_data)
